# baseline (speedup 1.0000x reference)
import functools
import jax
import jax.numpy as jnp
from jax import lax
from jax.experimental import pallas as pl
from jax.experimental.pallas import tpu as pltpu
from jax.experimental.pallas import tpu_sc as plsc

NN = 10000
EE = 320000
TRASH = 2 * NN
ACC_ROWS = 2 * NN + 128
ACC2 = 20096
CCPAD = 10240
HALF = 64
QC = 32
ARPT = ACC2 // 16
FLEN = 624
NC, NS, LN = 2, 16, 16
NW = NC * NS
EPAD = 327680
EP_A = EPAD // NW
EP_B = EPAD // NS
CH = 128
NCH = EP_B // CH
GI = 16
NSG = NCH // GI
NB = 7
ROWS_PER_TILE = ACC_ROWS // NS
RT = 2000
NG = NN // RT


@functools.cache
def _mesh():
    return plsc.VectorSubcoreMesh(
        core_axis_name="c", subcore_axis_name="s",
        num_cores=NC, num_subcores=NS)


def _pass_a_body(row_hbm, col_hbm, cc_hbm, t_hbm, hist_hbm,
                 row_v, col_v, cc_v, t_v, hist_v):
    wid = lax.axis_index("s") * NC + lax.axis_index("c")
    base = wid * EP_A
    pltpu.sync_copy(row_hbm.at[pl.ds(base, EP_A)], row_v)
    pltpu.sync_copy(col_hbm.at[pl.ds(base, EP_A)], col_v)
    pltpu.sync_copy(cc_hbm, cc_v.at[pl.ds(0, NN)])

    def zero_step(i, carry):
        hist_v[pl.ds(i * LN, LN)] = jnp.zeros((LN,), jnp.float32)
        return carry
    lax.fori_loop(0, ACC_ROWS // LN, zero_step, 0)

    ones = jnp.ones((LN,), jnp.float32)
    trash = jnp.full((LN,), TRASH, jnp.int32)
    npad = jnp.full((LN,), NN, jnp.int32)
    zero = jnp.zeros((LN,), jnp.int32)

    def step(i, carry):
        r = row_v[pl.ds(i * LN, LN)]
        c = col_v[pl.ds(i * LN, LN)]
        ccv = plsc.load_gather(cc_v, [c])
        cls = jnp.where(ccv > 0.5, npad, zero)
        t = jnp.where(r != c, r + cls, trash)
        t_v[pl.ds(i * LN, LN)] = t
        plsc.addupdate_scatter(hist_v, [t], ones)
        return carry
    lax.fori_loop(0, EP_A // LN, step, 0)

    pltpu.sync_copy(t_v, t_hbm.at[pl.ds(base, EP_A)])
    pltpu.sync_copy(hist_v, hist_hbm.at[wid])


@functools.cache
def _pass_a():
    return pl.kernel(
        _pass_a_body,
        out_type=[jax.ShapeDtypeStruct((EPAD,), jnp.int32),
                  jax.ShapeDtypeStruct((NW, ACC_ROWS), jnp.float32)],
        mesh=_mesh(),
        compiler_params=pltpu.CompilerParams(needs_layout_passes=False),
        scratch_types=[pltpu.VMEM((EP_A,), jnp.int32),
                       pltpu.VMEM((EP_A,), jnp.int32),
                       pltpu.VMEM((CCPAD,), jnp.float32),
                       pltpu.VMEM((EP_A,), jnp.int32),
                       pltpu.VMEM((ACC_ROWS,), jnp.float32)])


def _spmm_kernel_body(t_hbm, col_hbm, feat_hbm, out_hbm,
                      tall, call, rbuf, feat_sp, acc, sem_g, sem_i, sem_s):
    cid = lax.axis_index("c")
    sid = lax.axis_index("s")
    FG = NB - 2

    row_off = sid * ARPT
    nfull = ARPT // CH
    rem = ARPT % CH

    def _load_idx_group(sg, half):
        pltpu.async_copy(t_hbm.at[sid, pl.ds(sg * GI, GI)],
                         tall.at[pl.ds(half, GI)], sem_i)
        pltpu.async_copy(col_hbm.at[sid, pl.ds(sg * GI, GI)],
                         call.at[pl.ds(half, GI)], sem_i)

    def _wait_idx_group(sg, half):
        pltpu.make_async_copy(t_hbm.at[sid, pl.ds(sg * GI, GI)],
                              tall.at[pl.ds(half, GI)], sem_i).wait()
        pltpu.make_async_copy(col_hbm.at[sid, pl.ds(sg * GI, GI)],
                              call.at[pl.ds(half, GI)], sem_i).wait()

    for q in (0, 1):
        gq = 2 * cid + q
        last = NN - 15 * FLEN

        @pl.when(sid < 15)
        def _():
            pltpu.sync_copy(feat_hbm.at[gq, pl.ds(sid * FLEN, FLEN)],
                            feat_sp.at[pl.ds(sid * FLEN, FLEN)])

        @pl.when(sid == 15)
        def _():
            pltpu.sync_copy(feat_hbm.at[gq, pl.ds(15 * FLEN, last)],
                            feat_sp.at[pl.ds(15 * FLEN, last)])

        def zrow(i, carry):
            def zcol(j, carry2):
                rbuf[0, i, pl.ds(j * LN, LN)] = jnp.zeros((LN,), jnp.float32)
                return carry2
            lax.fori_loop(0, QC // LN, zcol, 0)
            return carry
        lax.fori_loop(0, CH, zrow, 0)

        def zacc(i, carry):
            pltpu.sync_copy(rbuf.at[0], acc.at[pl.ds(row_off + i * CH, CH)])
            return carry
        lax.fori_loop(0, nfull, zacc, 0)
        pltpu.sync_copy(rbuf.at[0, pl.ds(0, rem)],
                        acc.at[pl.ds(row_off + nfull * CH, rem)])
        plsc.subcore_barrier()

        _load_idx_group(0, 0)
        _wait_idx_group(0, 0)
        for i in range(FG):
            pltpu.async_copy(feat_sp.at[call.at[i]], rbuf.at[i], sem_g)

        def sg_body(S, carry):
            hs = lax.rem(S, 2) * GI
            ns = lax.rem(S + 1, 2) * GI

            @pl.when(S >= 1)
            def _():
                for k in (GI - 2, GI - 1):
                    g_prev = (S - 1) * GI + k
                    pltpu.make_async_copy(rbuf.at[lax.rem(g_prev, NB)],
                                          acc.at[tall.at[ns + k]],
                                          sem_s).wait()

            @pl.when(S + 1 < NSG)
            def _():
                _load_idx_group(S + 1, ns)
            for i in range(GI):
                g = S * GI + i
                if i == 8:
                    @pl.when(S + 1 < NSG)
                    def _():
                        _wait_idx_group(S + 1, ns)
                if i >= 2:
                    pltpu.make_async_copy(rbuf.at[lax.rem(g - 2, NB)],
                                          acc.at[tall.at[hs + i - 2]],
                                          sem_s).wait()
                nrow = (hs + i + FG) if i + FG < GI else (ns + i + FG - GI)

                @pl.when(g + FG < NCH)
                def _():
                    pltpu.async_copy(feat_sp.at[call.at[nrow]],
                                     rbuf.at[lax.rem(g + FG, NB)], sem_g)
                pltpu.make_async_copy(feat_sp.at[call.at[hs + i]],
                                      rbuf.at[lax.rem(g, NB)], sem_g).wait()
                pltpu.async_copy(rbuf.at[lax.rem(g, NB)],
                                 acc.at[tall.at[hs + i]], sem_s, add=True)
            return carry
        lax.fori_loop(0, NSG, sg_body, 0)
        hs_last = ((NSG - 1) % 2) * GI
        for k in (GI - 2, GI - 1):
            g_last = (NSG - 1) * GI + k
            pltpu.make_async_copy(rbuf.at[g_last % NB],
                                  acc.at[tall.at[hs_last + k]], sem_s).wait()
        plsc.subcore_barrier()

        def wb(i, carry):
            pltpu.sync_copy(acc.at[pl.ds(row_off + i * CH, CH)],
                            out_hbm.at[gq, pl.ds(row_off + i * CH, CH)])
            return carry
        lax.fori_loop(0, nfull, wb, 0)
        pltpu.sync_copy(acc.at[pl.ds(row_off + nfull * CH, rem)],
                        out_hbm.at[gq, pl.ds(row_off + nfull * CH, rem)])


@functools.cache
def _spmm():
    return pl.kernel(
        _spmm_kernel_body,
        out_type=jax.ShapeDtypeStruct((4, ACC2, QC), jnp.float32),
        mesh=_mesh(),
        compiler_params=pltpu.CompilerParams(
            needs_layout_passes=False, use_tc_tiling_on_sc=False),
        scratch_types=[pltpu.VMEM((2 * GI, CH), jnp.int32),
                       pltpu.VMEM((2 * GI, CH), jnp.int32),
                       pltpu.VMEM((NB, CH, QC), jnp.float32),
                       pltpu.VMEM_SHARED((NN, QC), jnp.float32),
                       pltpu.VMEM_SHARED((ACC2, QC), jnp.float32),
                       pltpu.SemaphoreType.DMA,
                       pltpu.SemaphoreType.DMA,
                       pltpu.SemaphoreType.DMA])


def _tc1_body(x_ref, cc_ref, hist_ref, wx_ref, lam1_ref, lam2_ref,
              xg2_ref, xw_ref, scal_ref, smix_ref):
    cnt = jnp.sum(hist_ref[...], axis=0)
    cnt0 = cnt[0:NN]
    cnt1 = cnt[NN:2 * NN]
    rc = cnt0 + cnt1
    inv_denom = 1.0 / jnp.maximum(rc, 1.0)
    g = jnp.where(rc > 0, lax.rsqrt(jnp.maximum(rc, 1.0)), 0.0)
    cc = cc_ref[...]
    rev = 1.0 - cc
    a = cc * g
    b = rev * g
    p = jnp.where(cnt1 > 0, lax.rsqrt(jnp.maximum(cnt1, 1.0)), 0.0)
    q = jnp.where(cnt0 > 0, lax.rsqrt(jnp.maximum(cnt0, 1.0)), 0.0)
    x = x_ref[...]
    xg = g[:, None] * x
    for k in range(4):
        xg2_ref[k, :, :] = xg[:, k * QC:(k + 1) * QC]
    xw_ref[...] = jnp.dot(x, wx_ref[...], preferred_element_type=jnp.float32)
    scal_ref[:, 0] = a
    scal_ref[:, 1] = b
    scal_ref[:, 2] = p
    scal_ref[:, 3] = q
    scal_ref[:, 4] = inv_denom
    scal_ref[:, 5] = cc
    e1 = jnp.exp(lam1_ref[...])
    s1w = e1 / jnp.sum(e1)
    e2 = jnp.exp(lam2_ref[...])
    s2w = e2 / jnp.sum(e2)
    ii = lax.broadcasted_iota(jnp.int32, (1, 128), 1)
    smix = jnp.where(ii == 0, s1w[0],
                     jnp.where(ii == 1, s1w[1],
                               jnp.where(ii == 2, s2w[0],
                                         jnp.where(ii == 3, s2w[1], 0.0))))
    smix_ref[...] = smix


def _tc1(x, cc, hist, wx, lam1, lam2):
    return pl.pallas_call(
        _tc1_body,
        out_shape=[jax.ShapeDtypeStruct((4, NN, QC), jnp.float32),
                   jax.ShapeDtypeStruct((NN, 2 * HALF), jnp.float32),
                   jax.ShapeDtypeStruct((NN, 8), jnp.float32),
                   jax.ShapeDtypeStruct((1, 128), jnp.float32)],
    )(x, cc, hist, wx, lam1, lam2)


def _tc2_body(slo_ref, shi_ref, x_ref, scal_ref, smix_ref,
              w1l_ref, w1r_ref, w3l_ref, w3r_ref,
              w2l_ref, w2r_ref, w4l_ref, w4r_ref,
              uq2_ref, r2_ref):
    a = scal_ref[:, 0]
    b = scal_ref[:, 1]
    p = scal_ref[:, 2]
    q = scal_ref[:, 3]
    inv_denom = scal_ref[:, 4]
    cc = scal_ref[:, 5]
    s0 = jnp.concatenate([slo_ref[k] for k in range(4)], axis=1)
    s1 = jnp.concatenate([shi_ref[k] for k in range(4)], axis=1)
    agg1 = (a * inv_denom)[:, None] * s1
    agg3 = (b * inv_denom)[:, None] * s0
    x = x_ref[...]
    xl = jnp.maximum(
        jnp.dot(agg1, w1l_ref[...], preferred_element_type=jnp.float32)
        + jnp.dot(x, w1r_ref[...], preferred_element_type=jnp.float32), 0.0)
    xh = jnp.maximum(
        jnp.dot(agg3, w3l_ref[...], preferred_element_type=jnp.float32)
        + jnp.dot(x, w3r_ref[...], preferred_element_type=jnp.float32), 0.0)
    s11 = smix_ref[0, 1]
    s21 = smix_ref[0, 3]
    y2 = jnp.dot(xl, w2l_ref[...], preferred_element_type=jnp.float32)
    y4 = jnp.dot(xh, w4l_ref[...], preferred_element_type=jnp.float32)
    uq = ((s11 * cc * p)[:, None] * y2
          + (s21 * (1.0 - cc) * q)[:, None] * y4)
    for k in range(4):
        uq2_ref[k, :, :] = uq[:, k * QC:(k + 1) * QC]
    r2_ref[...] = (
        s11 * jnp.dot(xl, w2r_ref[...], preferred_element_type=jnp.float32)
        + s21 * jnp.dot(xh, w4r_ref[...], preferred_element_type=jnp.float32))


def _tc2(s, x, scal, smix, W1l, W1r, W3l, W3r, W2l, W2r, W4l, W4r):
    wspec = pl.BlockSpec((128, 128), lambda i: (0, 0))
    return pl.pallas_call(
        _tc2_body,
        grid=(NG,),
        in_specs=[
            pl.BlockSpec((4, RT, QC), lambda i: (0, i, 0)),
            pl.BlockSpec((4, RT, QC), lambda i: (0, i + NG, 0)),
            pl.BlockSpec((RT, 2 * HALF), lambda i: (i, 0)),
            pl.BlockSpec((RT, 8), lambda i: (i, 0)),
            pl.BlockSpec((1, 128), lambda i: (0, 0)),
            wspec, wspec, wspec, wspec, wspec, wspec, wspec, wspec,
        ],
        out_specs=[
            pl.BlockSpec((4, RT, QC), lambda i: (0, i, 0)),
            pl.BlockSpec((RT, 2 * HALF), lambda i: (i, 0)),
        ],
        out_shape=[jax.ShapeDtypeStruct((4, NN, QC), jnp.float32),
                   jax.ShapeDtypeStruct((NN, 2 * HALF), jnp.float32)],
    )(s, s, x, scal, smix, W1l, W1r, W3l, W3r, W2l, W2r, W4l, W4r)


def _tc3_body(tlo_ref, thi_ref, scal_ref, smix_ref, xw_ref, r2_ref,
              lw_ref, lb_ref, out_ref):
    p = scal_ref[:, 2]
    q = scal_ref[:, 3]
    inv_denom = scal_ref[:, 4]
    cc = scal_ref[:, 5]
    t0 = jnp.concatenate([tlo_ref[k] for k in range(4)], axis=1)
    t1 = jnp.concatenate([thi_ref[k] for k in range(4)], axis=1)
    comb2 = ((p * inv_denom)[:, None] * t1 + (q * inv_denom)[:, None] * t0)
    s10 = smix_ref[0, 0]
    s20 = smix_ref[0, 2]
    lamx = s10 * cc + s20 * (1.0 - cc)
    xf = jnp.maximum(lamx[:, None] * xw_ref[...] + comb2 + r2_ref[...], 0.0)
    out_ref[...] = (jnp.dot(xf, lw_ref[...], preferred_element_type=jnp.float32)
                    + lb_ref[...])


def _tc3(t, scal, smix, xw, r2, lin1_W, lin1_b):
    return pl.pallas_call(
        _tc3_body,
        grid=(NG,),
        in_specs=[
            pl.BlockSpec((4, RT, QC), lambda i: (0, i, 0)),
            pl.BlockSpec((4, RT, QC), lambda i: (0, i + NG, 0)),
            pl.BlockSpec((RT, 8), lambda i: (i, 0)),
            pl.BlockSpec((1, 128), lambda i: (0, 0)),
            pl.BlockSpec((RT, 2 * HALF), lambda i: (i, 0)),
            pl.BlockSpec((RT, 2 * HALF), lambda i: (i, 0)),
            pl.BlockSpec((128, 64), lambda i: (0, 0)),
            pl.BlockSpec((1, 64), lambda i: (0, 0)),
        ],
        out_specs=pl.BlockSpec((RT, 64), lambda i: (i, 0)),
        out_shape=jax.ShapeDtypeStruct((NN, 64), jnp.float32),
    )(t, t, scal, smix, xw, r2, lin1_W, lin1_b)


@jax.jit
def kernel(x, cc_mask, W1l, W1r, W2l, W2r, W3l, W3r, W4l, W4r, WX,
           lam1, lam2, lin1_W, lin1_b, edge_index):
    row = edge_index[1]
    col = edge_index[0]
    padz = jnp.zeros((EPAD - EE,), jnp.int32)
    row_p = jnp.concatenate([row, padz])
    col_p = jnp.concatenate([col, padz])

    t_idx, hist = _pass_a()(row_p, col_p, cc_mask)
    xg2, xw, scal, smix = _tc1(x, cc_mask, hist, WX, lam1, lam2)
    t3 = t_idx.reshape(NS, NCH, CH)
    c3 = col_p.reshape(NS, NCH, CH)
    s = _spmm()(t3, c3, xg2)
    uq2, r2 = _tc2(s, x, scal, smix, W1l, W1r, W3l, W3r, W2l, W2r, W4l, W4r)
    t = _spmm()(t3, c3, uq2)
    return _tc3(t, scal, smix, xw, r2, lin1_W, lin1_b.reshape(1, 64))


# speedup vs baseline: 1.0004x; 1.0004x over previous
import functools
import jax
import jax.numpy as jnp
from jax import lax
from jax.experimental import pallas as pl
from jax.experimental.pallas import tpu as pltpu
from jax.experimental.pallas import tpu_sc as plsc

NN = 10000
EE = 320000
TRASH = 2 * NN
ACC_ROWS = 2 * NN + 128
ACC2 = 20096
CCPAD = 10240
HALF = 64
QC = 32
ARPT = ACC2 // 16
FLEN = 624
NC, NS, LN = 2, 16, 16
NW = NC * NS
EPAD = 327680
EP_A = EPAD // NW
EP_B = EPAD // NS
CH = 128
NCH = EP_B // CH
GI = 32
NSG = NCH // GI
NB = 5
ROWS_PER_TILE = ACC_ROWS // NS
RT = 2000
NG = NN // RT


@functools.cache
def _mesh():
    return plsc.VectorSubcoreMesh(
        core_axis_name="c", subcore_axis_name="s",
        num_cores=NC, num_subcores=NS)


def _pass_a_body(row_hbm, col_hbm, cc_hbm, t_hbm, hist_hbm,
                 row_v, col_v, cc_v, t_v, hist_v):
    wid = lax.axis_index("s") * NC + lax.axis_index("c")
    base = wid * EP_A
    pltpu.sync_copy(row_hbm.at[pl.ds(base, EP_A)], row_v)
    pltpu.sync_copy(col_hbm.at[pl.ds(base, EP_A)], col_v)
    pltpu.sync_copy(cc_hbm, cc_v.at[pl.ds(0, NN)])

    def zero_step(i, carry):
        hist_v[pl.ds(i * LN, LN)] = jnp.zeros((LN,), jnp.float32)
        return carry
    lax.fori_loop(0, ACC_ROWS // LN, zero_step, 0)

    ones = jnp.ones((LN,), jnp.float32)
    trash = jnp.full((LN,), TRASH, jnp.int32)
    npad = jnp.full((LN,), NN, jnp.int32)
    zero = jnp.zeros((LN,), jnp.int32)

    def step(i, carry):
        r = row_v[pl.ds(i * LN, LN)]
        c = col_v[pl.ds(i * LN, LN)]
        ccv = plsc.load_gather(cc_v, [c])
        cls = jnp.where(ccv > 0.5, npad, zero)
        t = jnp.where(r != c, r + cls, trash)
        t_v[pl.ds(i * LN, LN)] = t
        plsc.addupdate_scatter(hist_v, [t], ones)
        return carry
    lax.fori_loop(0, EP_A // LN, step, 0)

    pltpu.sync_copy(t_v, t_hbm.at[pl.ds(base, EP_A)])
    pltpu.sync_copy(hist_v, hist_hbm.at[wid])


@functools.cache
def _pass_a():
    return pl.kernel(
        _pass_a_body,
        out_type=[jax.ShapeDtypeStruct((EPAD,), jnp.int32),
                  jax.ShapeDtypeStruct((NW, ACC_ROWS), jnp.float32)],
        mesh=_mesh(),
        compiler_params=pltpu.CompilerParams(needs_layout_passes=False),
        scratch_types=[pltpu.VMEM((EP_A,), jnp.int32),
                       pltpu.VMEM((EP_A,), jnp.int32),
                       pltpu.VMEM((CCPAD,), jnp.float32),
                       pltpu.VMEM((EP_A,), jnp.int32),
                       pltpu.VMEM((ACC_ROWS,), jnp.float32)])


def _spmm_kernel_body(t_hbm, col_hbm, feat_hbm, out_hbm,
                      tall, call, rbuf, feat_sp, acc, sem_g, sem_i, sem_s):
    cid = lax.axis_index("c")
    sid = lax.axis_index("s")
    FG = NB - 2

    row_off = sid * ARPT
    nfull = ARPT // CH
    rem = ARPT % CH

    def _load_idx_group(sg, half):
        pltpu.async_copy(t_hbm.at[sid, pl.ds(sg * GI, GI)],
                         tall.at[pl.ds(half, GI)], sem_i)
        pltpu.async_copy(col_hbm.at[sid, pl.ds(sg * GI, GI)],
                         call.at[pl.ds(half, GI)], sem_i)

    def _wait_idx_group(sg, half):
        pltpu.make_async_copy(t_hbm.at[sid, pl.ds(sg * GI, GI)],
                              tall.at[pl.ds(half, GI)], sem_i).wait()
        pltpu.make_async_copy(col_hbm.at[sid, pl.ds(sg * GI, GI)],
                              call.at[pl.ds(half, GI)], sem_i).wait()

    for q in (0, 1):
        gq = 2 * cid + q
        last = NN - 15 * FLEN

        @pl.when(sid < 15)
        def _():
            pltpu.sync_copy(feat_hbm.at[gq, pl.ds(sid * FLEN, FLEN)],
                            feat_sp.at[pl.ds(sid * FLEN, FLEN)])

        @pl.when(sid == 15)
        def _():
            pltpu.sync_copy(feat_hbm.at[gq, pl.ds(15 * FLEN, last)],
                            feat_sp.at[pl.ds(15 * FLEN, last)])

        def zrow(i, carry):
            def zcol(j, carry2):
                rbuf[0, i, pl.ds(j * LN, LN)] = jnp.zeros((LN,), jnp.float32)
                return carry2
            lax.fori_loop(0, QC // LN, zcol, 0)
            return carry
        lax.fori_loop(0, CH, zrow, 0)

        def zacc(i, carry):
            pltpu.sync_copy(rbuf.at[0], acc.at[pl.ds(row_off + i * CH, CH)])
            return carry
        lax.fori_loop(0, nfull, zacc, 0)
        pltpu.sync_copy(rbuf.at[0, pl.ds(0, rem)],
                        acc.at[pl.ds(row_off + nfull * CH, rem)])
        plsc.subcore_barrier()

        _load_idx_group(0, 0)
        _wait_idx_group(0, 0)
        for i in range(FG):
            pltpu.async_copy(feat_sp.at[call.at[i]], rbuf.at[i], sem_g)

        def sg_body(S, carry):
            hs = lax.rem(S, 2) * GI
            ns = lax.rem(S + 1, 2) * GI

            @pl.when(S >= 1)
            def _():
                for k in (GI - 2, GI - 1):
                    g_prev = (S - 1) * GI + k
                    pltpu.make_async_copy(rbuf.at[lax.rem(g_prev, NB)],
                                          acc.at[tall.at[ns + k]],
                                          sem_s).wait()

            @pl.when(S + 1 < NSG)
            def _():
                _load_idx_group(S + 1, ns)
            for i in range(GI):
                g = S * GI + i
                if i == 8:
                    @pl.when(S + 1 < NSG)
                    def _():
                        _wait_idx_group(S + 1, ns)
                if i >= 2:
                    pltpu.make_async_copy(rbuf.at[lax.rem(g - 2, NB)],
                                          acc.at[tall.at[hs + i - 2]],
                                          sem_s).wait()
                nrow = (hs + i + FG) if i + FG < GI else (ns + i + FG - GI)

                @pl.when(g + FG < NCH)
                def _():
                    pltpu.async_copy(feat_sp.at[call.at[nrow]],
                                     rbuf.at[lax.rem(g + FG, NB)], sem_g)
                pltpu.make_async_copy(feat_sp.at[call.at[hs + i]],
                                      rbuf.at[lax.rem(g, NB)], sem_g).wait()
                pltpu.async_copy(rbuf.at[lax.rem(g, NB)],
                                 acc.at[tall.at[hs + i]], sem_s, add=True)
            return carry
        lax.fori_loop(0, NSG, sg_body, 0)
        hs_last = ((NSG - 1) % 2) * GI
        for k in (GI - 2, GI - 1):
            g_last = (NSG - 1) * GI + k
            pltpu.make_async_copy(rbuf.at[g_last % NB],
                                  acc.at[tall.at[hs_last + k]], sem_s).wait()
        plsc.subcore_barrier()

        def wb(i, carry):
            pltpu.sync_copy(acc.at[pl.ds(row_off + i * CH, CH)],
                            out_hbm.at[gq, pl.ds(row_off + i * CH, CH)])
            return carry
        lax.fori_loop(0, nfull, wb, 0)
        pltpu.sync_copy(acc.at[pl.ds(row_off + nfull * CH, rem)],
                        out_hbm.at[gq, pl.ds(row_off + nfull * CH, rem)])


@functools.cache
def _spmm():
    return pl.kernel(
        _spmm_kernel_body,
        out_type=jax.ShapeDtypeStruct((4, ACC2, QC), jnp.float32),
        mesh=_mesh(),
        compiler_params=pltpu.CompilerParams(
            needs_layout_passes=False, use_tc_tiling_on_sc=False),
        scratch_types=[pltpu.VMEM((2 * GI, CH), jnp.int32),
                       pltpu.VMEM((2 * GI, CH), jnp.int32),
                       pltpu.VMEM((NB, CH, QC), jnp.float32),
                       pltpu.VMEM_SHARED((NN, QC), jnp.float32),
                       pltpu.VMEM_SHARED((ACC2, QC), jnp.float32),
                       pltpu.SemaphoreType.DMA,
                       pltpu.SemaphoreType.DMA,
                       pltpu.SemaphoreType.DMA])


def _tc1_body(x_ref, cc_ref, hist_ref, wx_ref, lam1_ref, lam2_ref,
              xg2_ref, xw_ref, scal_ref, smix_ref):
    cnt = jnp.sum(hist_ref[...], axis=0)
    cnt0 = cnt[0:NN]
    cnt1 = cnt[NN:2 * NN]
    rc = cnt0 + cnt1
    inv_denom = 1.0 / jnp.maximum(rc, 1.0)
    g = jnp.where(rc > 0, lax.rsqrt(jnp.maximum(rc, 1.0)), 0.0)
    cc = cc_ref[...]
    rev = 1.0 - cc
    a = cc * g
    b = rev * g
    p = jnp.where(cnt1 > 0, lax.rsqrt(jnp.maximum(cnt1, 1.0)), 0.0)
    q = jnp.where(cnt0 > 0, lax.rsqrt(jnp.maximum(cnt0, 1.0)), 0.0)
    x = x_ref[...]
    xg = g[:, None] * x
    for k in range(4):
        xg2_ref[k, :, :] = xg[:, k * QC:(k + 1) * QC]
    xw_ref[...] = jnp.dot(x, wx_ref[...], preferred_element_type=jnp.float32)
    scal_ref[:, 0] = a
    scal_ref[:, 1] = b
    scal_ref[:, 2] = p
    scal_ref[:, 3] = q
    scal_ref[:, 4] = inv_denom
    scal_ref[:, 5] = cc
    e1 = jnp.exp(lam1_ref[...])
    s1w = e1 / jnp.sum(e1)
    e2 = jnp.exp(lam2_ref[...])
    s2w = e2 / jnp.sum(e2)
    ii = lax.broadcasted_iota(jnp.int32, (1, 128), 1)
    smix = jnp.where(ii == 0, s1w[0],
                     jnp.where(ii == 1, s1w[1],
                               jnp.where(ii == 2, s2w[0],
                                         jnp.where(ii == 3, s2w[1], 0.0))))
    smix_ref[...] = smix


def _tc1(x, cc, hist, wx, lam1, lam2):
    return pl.pallas_call(
        _tc1_body,
        out_shape=[jax.ShapeDtypeStruct((4, NN, QC), jnp.float32),
                   jax.ShapeDtypeStruct((NN, 2 * HALF), jnp.float32),
                   jax.ShapeDtypeStruct((NN, 8), jnp.float32),
                   jax.ShapeDtypeStruct((1, 128), jnp.float32)],
    )(x, cc, hist, wx, lam1, lam2)


def _tc2_body(slo_ref, shi_ref, x_ref, scal_ref, smix_ref,
              w1l_ref, w1r_ref, w3l_ref, w3r_ref,
              w2l_ref, w2r_ref, w4l_ref, w4r_ref,
              uq2_ref, r2_ref):
    a = scal_ref[:, 0]
    b = scal_ref[:, 1]
    p = scal_ref[:, 2]
    q = scal_ref[:, 3]
    inv_denom = scal_ref[:, 4]
    cc = scal_ref[:, 5]
    s0 = jnp.concatenate([slo_ref[k] for k in range(4)], axis=1)
    s1 = jnp.concatenate([shi_ref[k] for k in range(4)], axis=1)
    agg1 = (a * inv_denom)[:, None] * s1
    agg3 = (b * inv_denom)[:, None] * s0
    x = x_ref[...]
    xl = jnp.maximum(
        jnp.dot(agg1, w1l_ref[...], preferred_element_type=jnp.float32)
        + jnp.dot(x, w1r_ref[...], preferred_element_type=jnp.float32), 0.0)
    xh = jnp.maximum(
        jnp.dot(agg3, w3l_ref[...], preferred_element_type=jnp.float32)
        + jnp.dot(x, w3r_ref[...], preferred_element_type=jnp.float32), 0.0)
    s11 = smix_ref[0, 1]
    s21 = smix_ref[0, 3]
    y2 = jnp.dot(xl, w2l_ref[...], preferred_element_type=jnp.float32)
    y4 = jnp.dot(xh, w4l_ref[...], preferred_element_type=jnp.float32)
    uq = ((s11 * cc * p)[:, None] * y2
          + (s21 * (1.0 - cc) * q)[:, None] * y4)
    for k in range(4):
        uq2_ref[k, :, :] = uq[:, k * QC:(k + 1) * QC]
    r2_ref[...] = (
        s11 * jnp.dot(xl, w2r_ref[...], preferred_element_type=jnp.float32)
        + s21 * jnp.dot(xh, w4r_ref[...], preferred_element_type=jnp.float32))


def _tc2(s, x, scal, smix, W1l, W1r, W3l, W3r, W2l, W2r, W4l, W4r):
    wspec = pl.BlockSpec((128, 128), lambda i: (0, 0))
    return pl.pallas_call(
        _tc2_body,
        grid=(NG,),
        in_specs=[
            pl.BlockSpec((4, RT, QC), lambda i: (0, i, 0)),
            pl.BlockSpec((4, RT, QC), lambda i: (0, i + NG, 0)),
            pl.BlockSpec((RT, 2 * HALF), lambda i: (i, 0)),
            pl.BlockSpec((RT, 8), lambda i: (i, 0)),
            pl.BlockSpec((1, 128), lambda i: (0, 0)),
            wspec, wspec, wspec, wspec, wspec, wspec, wspec, wspec,
        ],
        out_specs=[
            pl.BlockSpec((4, RT, QC), lambda i: (0, i, 0)),
            pl.BlockSpec((RT, 2 * HALF), lambda i: (i, 0)),
        ],
        out_shape=[jax.ShapeDtypeStruct((4, NN, QC), jnp.float32),
                   jax.ShapeDtypeStruct((NN, 2 * HALF), jnp.float32)],
    )(s, s, x, scal, smix, W1l, W1r, W3l, W3r, W2l, W2r, W4l, W4r)


def _tc3_body(tlo_ref, thi_ref, scal_ref, smix_ref, xw_ref, r2_ref,
              lw_ref, lb_ref, out_ref):
    p = scal_ref[:, 2]
    q = scal_ref[:, 3]
    inv_denom = scal_ref[:, 4]
    cc = scal_ref[:, 5]
    t0 = jnp.concatenate([tlo_ref[k] for k in range(4)], axis=1)
    t1 = jnp.concatenate([thi_ref[k] for k in range(4)], axis=1)
    comb2 = ((p * inv_denom)[:, None] * t1 + (q * inv_denom)[:, None] * t0)
    s10 = smix_ref[0, 0]
    s20 = smix_ref[0, 2]
    lamx = s10 * cc + s20 * (1.0 - cc)
    xf = jnp.maximum(lamx[:, None] * xw_ref[...] + comb2 + r2_ref[...], 0.0)
    out_ref[...] = (jnp.dot(xf, lw_ref[...], preferred_element_type=jnp.float32)
                    + lb_ref[...])


def _tc3(t, scal, smix, xw, r2, lin1_W, lin1_b):
    return pl.pallas_call(
        _tc3_body,
        grid=(NG,),
        in_specs=[
            pl.BlockSpec((4, RT, QC), lambda i: (0, i, 0)),
            pl.BlockSpec((4, RT, QC), lambda i: (0, i + NG, 0)),
            pl.BlockSpec((RT, 8), lambda i: (i, 0)),
            pl.BlockSpec((1, 128), lambda i: (0, 0)),
            pl.BlockSpec((RT, 2 * HALF), lambda i: (i, 0)),
            pl.BlockSpec((RT, 2 * HALF), lambda i: (i, 0)),
            pl.BlockSpec((128, 64), lambda i: (0, 0)),
            pl.BlockSpec((1, 64), lambda i: (0, 0)),
        ],
        out_specs=pl.BlockSpec((RT, 64), lambda i: (i, 0)),
        out_shape=jax.ShapeDtypeStruct((NN, 64), jnp.float32),
    )(t, t, scal, smix, xw, r2, lin1_W, lin1_b)


@jax.jit
def kernel(x, cc_mask, W1l, W1r, W2l, W2r, W3l, W3r, W4l, W4r, WX,
           lam1, lam2, lin1_W, lin1_b, edge_index):
    row = edge_index[1]
    col = edge_index[0]
    padz = jnp.zeros((EPAD - EE,), jnp.int32)
    row_p = jnp.concatenate([row, padz])
    col_p = jnp.concatenate([col, padz])

    t_idx, hist = _pass_a()(row_p, col_p, cc_mask)
    xg2, xw, scal, smix = _tc1(x, cc_mask, hist, WX, lam1, lam2)
    t3 = t_idx.reshape(NS, NCH, CH)
    c3 = col_p.reshape(NS, NCH, CH)
    s = _spmm()(t3, c3, xg2)
    uq2, r2 = _tc2(s, x, scal, smix, W1l, W1r, W3l, W3r, W2l, W2r, W4l, W4r)
    t = _spmm()(t3, c3, uq2)
    return _tc3(t, scal, smix, xw, r2, lin1_W, lin1_b.reshape(1, 64))


# speedup vs baseline: 1.0037x; 1.0033x over previous
import functools
import jax
import jax.numpy as jnp
from jax import lax
from jax.experimental import pallas as pl
from jax.experimental.pallas import tpu as pltpu
from jax.experimental.pallas import tpu_sc as plsc

NN = 10000
EE = 320000
TRASH = 2 * NN
ACC_ROWS = 2 * NN + 128
ACC2 = 20096
CCPAD = 10240
HALF = 64
QC = 32
ARPT = ACC2 // 16
FLEN = 624
NC, NS, LN = 2, 16, 16
NW = NC * NS
EPAD = 327680
EP_A = EPAD // NW
EP_B = EPAD // NS
CH = 128
NCH = EP_B // CH
GI = 16
NSG = NCH // GI
NB = 5
ROWS_PER_TILE = ACC_ROWS // NS
RT = 2000
NG = NN // RT


@functools.cache
def _mesh():
    return plsc.VectorSubcoreMesh(
        core_axis_name="c", subcore_axis_name="s",
        num_cores=NC, num_subcores=NS)


def _pass_a_body(row_hbm, col_hbm, cc_hbm, t_hbm, hist_hbm,
                 row_v, col_v, cc_v, t_v, hist_v):
    wid = lax.axis_index("s") * NC + lax.axis_index("c")
    base = wid * EP_A
    pltpu.sync_copy(row_hbm.at[pl.ds(base, EP_A)], row_v)
    pltpu.sync_copy(col_hbm.at[pl.ds(base, EP_A)], col_v)
    pltpu.sync_copy(cc_hbm, cc_v.at[pl.ds(0, NN)])

    def zero_step(i, carry):
        hist_v[pl.ds(i * LN, LN)] = jnp.zeros((LN,), jnp.float32)
        return carry
    lax.fori_loop(0, ACC_ROWS // LN, zero_step, 0)

    ones = jnp.ones((LN,), jnp.float32)
    trash = jnp.full((LN,), TRASH, jnp.int32)
    npad = jnp.full((LN,), NN, jnp.int32)
    zero = jnp.zeros((LN,), jnp.int32)

    def step(i, carry):
        r = row_v[pl.ds(i * LN, LN)]
        c = col_v[pl.ds(i * LN, LN)]
        ccv = plsc.load_gather(cc_v, [c])
        cls = jnp.where(ccv > 0.5, npad, zero)
        t = jnp.where(r != c, r + cls, trash)
        t_v[pl.ds(i * LN, LN)] = t
        plsc.addupdate_scatter(hist_v, [t], ones)
        return carry
    lax.fori_loop(0, EP_A // LN, step, 0)

    pltpu.sync_copy(t_v, t_hbm.at[pl.ds(base, EP_A)])
    pltpu.sync_copy(hist_v, hist_hbm.at[wid])


@functools.cache
def _pass_a():
    return pl.kernel(
        _pass_a_body,
        out_type=[jax.ShapeDtypeStruct((EPAD,), jnp.int32),
                  jax.ShapeDtypeStruct((NW, ACC_ROWS), jnp.float32)],
        mesh=_mesh(),
        compiler_params=pltpu.CompilerParams(needs_layout_passes=False),
        scratch_types=[pltpu.VMEM((EP_A,), jnp.int32),
                       pltpu.VMEM((EP_A,), jnp.int32),
                       pltpu.VMEM((CCPAD,), jnp.float32),
                       pltpu.VMEM((EP_A,), jnp.int32),
                       pltpu.VMEM((ACC_ROWS,), jnp.float32)])


def _spmm_kernel_body(t_hbm, col_hbm, feat_hbm, out_hbm,
                      tall, call, rbuf, feat_sp, acc, sem_g, sem_i, sem_s):
    cid = lax.axis_index("c")
    sid = lax.axis_index("s")
    FG = NB - 2

    row_off = sid * ARPT
    nfull = ARPT // CH
    rem = ARPT % CH

    def _load_idx_group(sg, half):
        pltpu.async_copy(t_hbm.at[sid, pl.ds(sg * GI, GI)],
                         tall.at[pl.ds(half, GI)], sem_i)
        pltpu.async_copy(col_hbm.at[sid, pl.ds(sg * GI, GI)],
                         call.at[pl.ds(half, GI)], sem_i)

    def _wait_idx_group(sg, half):
        pltpu.make_async_copy(t_hbm.at[sid, pl.ds(sg * GI, GI)],
                              tall.at[pl.ds(half, GI)], sem_i).wait()
        pltpu.make_async_copy(col_hbm.at[sid, pl.ds(sg * GI, GI)],
                              call.at[pl.ds(half, GI)], sem_i).wait()

    for q in (0, 1):
        gq = 2 * cid + q
        last = NN - 15 * FLEN

        @pl.when(sid < 15)
        def _():
            pltpu.sync_copy(feat_hbm.at[gq, pl.ds(sid * FLEN, FLEN)],
                            feat_sp.at[pl.ds(sid * FLEN, FLEN)])

        @pl.when(sid == 15)
        def _():
            pltpu.sync_copy(feat_hbm.at[gq, pl.ds(15 * FLEN, last)],
                            feat_sp.at[pl.ds(15 * FLEN, last)])

        def zrow(i, carry):
            def zcol(j, carry2):
                rbuf[0, i, pl.ds(j * LN, LN)] = jnp.zeros((LN,), jnp.float32)
                return carry2
            lax.fori_loop(0, QC // LN, zcol, 0)
            return carry
        lax.fori_loop(0, CH, zrow, 0)

        def zacc(i, carry):
            pltpu.sync_copy(rbuf.at[0], acc.at[pl.ds(row_off + i * CH, CH)])
            return carry
        lax.fori_loop(0, nfull, zacc, 0)
        pltpu.sync_copy(rbuf.at[0, pl.ds(0, rem)],
                        acc.at[pl.ds(row_off + nfull * CH, rem)])
        plsc.subcore_barrier()

        _load_idx_group(0, 0)
        _wait_idx_group(0, 0)
        for i in range(FG):
            pltpu.async_copy(feat_sp.at[call.at[i]], rbuf.at[i], sem_g)

        def sg_body(S, carry):
            hs = lax.rem(S, 2) * GI
            ns = lax.rem(S + 1, 2) * GI

            @pl.when(S >= 1)
            def _():
                for k in (GI - 2, GI - 1):
                    g_prev = (S - 1) * GI + k
                    pltpu.make_async_copy(rbuf.at[lax.rem(g_prev, NB)],
                                          acc.at[tall.at[ns + k]],
                                          sem_s).wait()

            @pl.when(S + 1 < NSG)
            def _():
                _load_idx_group(S + 1, ns)
            for i in range(GI):
                g = S * GI + i
                if i == 8:
                    @pl.when(S + 1 < NSG)
                    def _():
                        _wait_idx_group(S + 1, ns)
                if i >= 2:
                    pltpu.make_async_copy(rbuf.at[lax.rem(g - 2, NB)],
                                          acc.at[tall.at[hs + i - 2]],
                                          sem_s).wait()
                nrow = (hs + i + FG) if i + FG < GI else (ns + i + FG - GI)

                @pl.when(g + FG < NCH)
                def _():
                    pltpu.async_copy(feat_sp.at[call.at[nrow]],
                                     rbuf.at[lax.rem(g + FG, NB)], sem_g)
                pltpu.make_async_copy(feat_sp.at[call.at[hs + i]],
                                      rbuf.at[lax.rem(g, NB)], sem_g).wait()
                pltpu.async_copy(rbuf.at[lax.rem(g, NB)],
                                 acc.at[tall.at[hs + i]], sem_s, add=True)
            return carry
        lax.fori_loop(0, NSG, sg_body, 0)
        hs_last = ((NSG - 1) % 2) * GI
        for k in (GI - 2, GI - 1):
            g_last = (NSG - 1) * GI + k
            pltpu.make_async_copy(rbuf.at[g_last % NB],
                                  acc.at[tall.at[hs_last + k]], sem_s).wait()
        plsc.subcore_barrier()

        def wb(i, carry):
            pltpu.sync_copy(acc.at[pl.ds(row_off + i * CH, CH)],
                            out_hbm.at[gq, pl.ds(row_off + i * CH, CH)])
            return carry
        lax.fori_loop(0, nfull, wb, 0)
        pltpu.sync_copy(acc.at[pl.ds(row_off + nfull * CH, rem)],
                        out_hbm.at[gq, pl.ds(row_off + nfull * CH, rem)])


@functools.cache
def _spmm():
    return pl.kernel(
        _spmm_kernel_body,
        out_type=jax.ShapeDtypeStruct((4, ACC2, QC), jnp.float32),
        mesh=_mesh(),
        compiler_params=pltpu.CompilerParams(
            needs_layout_passes=False, use_tc_tiling_on_sc=False),
        scratch_types=[pltpu.VMEM((2 * GI, CH), jnp.int32),
                       pltpu.VMEM((2 * GI, CH), jnp.int32),
                       pltpu.VMEM((NB, CH, QC), jnp.float32),
                       pltpu.VMEM_SHARED((NN, QC), jnp.float32),
                       pltpu.VMEM_SHARED((ACC2, QC), jnp.float32),
                       pltpu.SemaphoreType.DMA,
                       pltpu.SemaphoreType.DMA,
                       pltpu.SemaphoreType.DMA])


def _tc1_body(x_ref, cc_ref, hist_ref, wx_ref, lam1_ref, lam2_ref,
              xg2_ref, xw_ref, scal_ref, smix_ref):
    cnt = jnp.sum(hist_ref[...], axis=0)
    cnt0 = cnt[0:NN]
    cnt1 = cnt[NN:2 * NN]
    rc = cnt0 + cnt1
    inv_denom = 1.0 / jnp.maximum(rc, 1.0)
    g = jnp.where(rc > 0, lax.rsqrt(jnp.maximum(rc, 1.0)), 0.0)
    cc = cc_ref[...]
    rev = 1.0 - cc
    a = cc * g
    b = rev * g
    p = jnp.where(cnt1 > 0, lax.rsqrt(jnp.maximum(cnt1, 1.0)), 0.0)
    q = jnp.where(cnt0 > 0, lax.rsqrt(jnp.maximum(cnt0, 1.0)), 0.0)
    x = x_ref[...]
    xg = g[:, None] * x
    for k in range(4):
        xg2_ref[k, :, :] = xg[:, k * QC:(k + 1) * QC]
    xw_ref[...] = jnp.dot(x, wx_ref[...], preferred_element_type=jnp.float32)
    scal_ref[:, 0] = a
    scal_ref[:, 1] = b
    scal_ref[:, 2] = p
    scal_ref[:, 3] = q
    scal_ref[:, 4] = inv_denom
    scal_ref[:, 5] = cc
    e1 = jnp.exp(lam1_ref[...])
    s1w = e1 / jnp.sum(e1)
    e2 = jnp.exp(lam2_ref[...])
    s2w = e2 / jnp.sum(e2)
    ii = lax.broadcasted_iota(jnp.int32, (1, 128), 1)
    smix = jnp.where(ii == 0, s1w[0],
                     jnp.where(ii == 1, s1w[1],
                               jnp.where(ii == 2, s2w[0],
                                         jnp.where(ii == 3, s2w[1], 0.0))))
    smix_ref[...] = smix


def _tc1(x, cc, hist, wx, lam1, lam2):
    return pl.pallas_call(
        _tc1_body,
        out_shape=[jax.ShapeDtypeStruct((4, NN, QC), jnp.float32),
                   jax.ShapeDtypeStruct((NN, 2 * HALF), jnp.float32),
                   jax.ShapeDtypeStruct((NN, 8), jnp.float32),
                   jax.ShapeDtypeStruct((1, 128), jnp.float32)],
    )(x, cc, hist, wx, lam1, lam2)


def _tc2_body(slo_ref, shi_ref, x_ref, scal_ref, smix_ref,
              w1l_ref, w1r_ref, w3l_ref, w3r_ref,
              w2l_ref, w2r_ref, w4l_ref, w4r_ref,
              uq2_ref, r2_ref):
    a = scal_ref[:, 0]
    b = scal_ref[:, 1]
    p = scal_ref[:, 2]
    q = scal_ref[:, 3]
    inv_denom = scal_ref[:, 4]
    cc = scal_ref[:, 5]
    s0 = jnp.concatenate([slo_ref[k] for k in range(4)], axis=1)
    s1 = jnp.concatenate([shi_ref[k] for k in range(4)], axis=1)
    agg1 = (a * inv_denom)[:, None] * s1
    agg3 = (b * inv_denom)[:, None] * s0
    x = x_ref[...]
    xl = jnp.maximum(
        jnp.dot(agg1, w1l_ref[...], preferred_element_type=jnp.float32)
        + jnp.dot(x, w1r_ref[...], preferred_element_type=jnp.float32), 0.0)
    xh = jnp.maximum(
        jnp.dot(agg3, w3l_ref[...], preferred_element_type=jnp.float32)
        + jnp.dot(x, w3r_ref[...], preferred_element_type=jnp.float32), 0.0)
    s11 = smix_ref[0, 1]
    s21 = smix_ref[0, 3]
    y2 = jnp.dot(xl, w2l_ref[...], preferred_element_type=jnp.float32)
    y4 = jnp.dot(xh, w4l_ref[...], preferred_element_type=jnp.float32)
    uq = ((s11 * cc * p)[:, None] * y2
          + (s21 * (1.0 - cc) * q)[:, None] * y4)
    for k in range(4):
        uq2_ref[k, :, :] = uq[:, k * QC:(k + 1) * QC]
    r2_ref[...] = (
        s11 * jnp.dot(xl, w2r_ref[...], preferred_element_type=jnp.float32)
        + s21 * jnp.dot(xh, w4r_ref[...], preferred_element_type=jnp.float32))


def _tc2(s, x, scal, smix, W1l, W1r, W3l, W3r, W2l, W2r, W4l, W4r):
    wspec = pl.BlockSpec((128, 128), lambda i: (0, 0))
    return pl.pallas_call(
        _tc2_body,
        grid=(NG,),
        in_specs=[
            pl.BlockSpec((4, RT, QC), lambda i: (0, i, 0)),
            pl.BlockSpec((4, RT, QC), lambda i: (0, i + NG, 0)),
            pl.BlockSpec((RT, 2 * HALF), lambda i: (i, 0)),
            pl.BlockSpec((RT, 8), lambda i: (i, 0)),
            pl.BlockSpec((1, 128), lambda i: (0, 0)),
            wspec, wspec, wspec, wspec, wspec, wspec, wspec, wspec,
        ],
        out_specs=[
            pl.BlockSpec((4, RT, QC), lambda i: (0, i, 0)),
            pl.BlockSpec((RT, 2 * HALF), lambda i: (i, 0)),
        ],
        out_shape=[jax.ShapeDtypeStruct((4, NN, QC), jnp.float32),
                   jax.ShapeDtypeStruct((NN, 2 * HALF), jnp.float32)],
    )(s, s, x, scal, smix, W1l, W1r, W3l, W3r, W2l, W2r, W4l, W4r)


def _tc3_body(tlo_ref, thi_ref, scal_ref, smix_ref, xw_ref, r2_ref,
              lw_ref, lb_ref, out_ref):
    p = scal_ref[:, 2]
    q = scal_ref[:, 3]
    inv_denom = scal_ref[:, 4]
    cc = scal_ref[:, 5]
    t0 = jnp.concatenate([tlo_ref[k] for k in range(4)], axis=1)
    t1 = jnp.concatenate([thi_ref[k] for k in range(4)], axis=1)
    comb2 = ((p * inv_denom)[:, None] * t1 + (q * inv_denom)[:, None] * t0)
    s10 = smix_ref[0, 0]
    s20 = smix_ref[0, 2]
    lamx = s10 * cc + s20 * (1.0 - cc)
    xf = jnp.maximum(lamx[:, None] * xw_ref[...] + comb2 + r2_ref[...], 0.0)
    out_ref[...] = (jnp.dot(xf, lw_ref[...], preferred_element_type=jnp.float32)
                    + lb_ref[...])


def _tc3(t, scal, smix, xw, r2, lin1_W, lin1_b):
    return pl.pallas_call(
        _tc3_body,
        grid=(NG,),
        in_specs=[
            pl.BlockSpec((4, RT, QC), lambda i: (0, i, 0)),
            pl.BlockSpec((4, RT, QC), lambda i: (0, i + NG, 0)),
            pl.BlockSpec((RT, 8), lambda i: (i, 0)),
            pl.BlockSpec((1, 128), lambda i: (0, 0)),
            pl.BlockSpec((RT, 2 * HALF), lambda i: (i, 0)),
            pl.BlockSpec((RT, 2 * HALF), lambda i: (i, 0)),
            pl.BlockSpec((128, 64), lambda i: (0, 0)),
            pl.BlockSpec((1, 64), lambda i: (0, 0)),
        ],
        out_specs=pl.BlockSpec((RT, 64), lambda i: (i, 0)),
        out_shape=jax.ShapeDtypeStruct((NN, 64), jnp.float32),
    )(t, t, scal, smix, xw, r2, lin1_W, lin1_b)


@jax.jit
def kernel(x, cc_mask, W1l, W1r, W2l, W2r, W3l, W3r, W4l, W4r, WX,
           lam1, lam2, lin1_W, lin1_b, edge_index):
    row = edge_index[1]
    col = edge_index[0]
    padz = jnp.zeros((EPAD - EE,), jnp.int32)
    row_p = jnp.concatenate([row, padz])
    col_p = jnp.concatenate([col, padz])

    t_idx, hist = _pass_a()(row_p, col_p, cc_mask)
    xg2, xw, scal, smix = _tc1(x, cc_mask, hist, WX, lam1, lam2)
    t3 = t_idx.reshape(NS, NCH, CH)
    c3 = col_p.reshape(NS, NCH, CH)
    s = _spmm()(t3, c3, xg2)
    uq2, r2 = _tc2(s, x, scal, smix, W1l, W1r, W3l, W3r, W2l, W2r, W4l, W4r)
    t = _spmm()(t3, c3, uq2)
    return _tc3(t, scal, smix, xw, r2, lin1_W, lin1_b.reshape(1, 64))
